# baseline (speedup 1.0000x reference)
import jax
import jax.numpy as jnp
from jax.experimental import pallas as pl
from jax.experimental.pallas import tpu as pltpu

_B, _S, _F, _H, _OUT = 4, 256, 32, 128, 1
_NBLK = 3
_HE = _H // 2
_TI = 16
_PAD = 10
_HI = jax.lax.Precision.HIGHEST


def _fused_kernel(data_ref, gd_ref, w11_ref, b11_ref, w12_ref, b12_ref,
                  w21_ref, b21_ref, w22_ref, b22_ref, w31_ref, b31_ref,
                  wf_ref, bf_ref, ws_ref, wr_ref, b1_ref, vblk_ref,
                  wl_ref, bl_ref, wrr_ref, bng_ref, bnb_ref,
                  gw_ref, we_ref, be_ref, wo_ref, bo_ref,
                  out_ref,
                  pad_ref, ps_ref, pr_ref, a_ref, h_ref, hn_ref):
    f32 = jnp.float32
    dd = data_ref[:].reshape(_B * _S, _F)

    t1 = jnp.dot(dd, w11_ref[:], preferred_element_type=f32, precision=_HI) + b11_ref[:]
    t2 = jnp.dot(dd, w21_ref[:], preferred_element_type=f32, precision=_HI) + b21_ref[:]
    t3 = jnp.dot(dd, w31_ref[:], preferred_element_type=f32, precision=_HI) + b31_ref[:]

    pad_ref[:] = jnp.zeros((_B, _S + 2 * _PAD, _H), f32)
    pad_ref[:, _PAD:_PAD + _S, :] = t1.reshape(_B, _S, _H)
    acc = None
    for k, off in enumerate((-3, 0, 3)):
        sh = pad_ref[:, _PAD + off:_PAD + off + _S, :].reshape(_B * _S, _H)
        p = jnp.dot(sh, w12_ref[k], preferred_element_type=f32, precision=_HI)
        acc = p if acc is None else acc + p
    x1 = acc + b12_ref[:]

    pad_ref[:, _PAD:_PAD + _S, :] = t2.reshape(_B, _S, _H)
    acc = None
    for k, off in enumerate((-10, -5, 0, 5, 10)):
        sh = pad_ref[:, _PAD + off:_PAD + off + _S, :].reshape(_B * _S, _H)
        p = jnp.dot(sh, w22_ref[k], preferred_element_type=f32, precision=_HI)
        acc = p if acc is None else acc + p
    x2 = acc + b22_ref[:]

    x = (jnp.dot(x1, wf_ref[0], preferred_element_type=f32, precision=_HI)
         + jnp.dot(x2, wf_ref[1], preferred_element_type=f32, precision=_HI)
         + jnp.dot(t3, wf_ref[2], preferred_element_type=f32, precision=_HI)
         + bf_ref[:])
    x = jnp.maximum(x, 0.0)

    ps_ref[:] = (jnp.dot(x, ws_ref[:], preferred_element_type=f32, precision=_HI)
                 + b1_ref[:]).reshape(_B, _S, _H)
    pr_ref[:] = jnp.dot(x, wr_ref[:],
                        preferred_element_type=f32,
                        precision=_HI).reshape(_B, _S // _TI,
                                                            _TI * _H)
    h_ref[:] = x.reshape(_B, _S, _H)

    a_ref[:] = jnp.zeros((_B, _S, _S), f32)
    for b in range(_B):
        psb = ps_ref[b]
        tps = jnp.concatenate([psb] * _TI, axis=1)
        for it in range(_S // _TI):
            r0 = it * _TI
            prrow = pr_ref[b, it:it + 1, :]
            ec = jnp.maximum(tps[r0:] + prrow, 0.0)
            sct = jnp.dot(ec, vblk_ref[:], precision=_HI,
                          preferred_element_type=f32)
            keep = sct + gd_ref[b, r0:, r0:r0 + _TI] >= 0.0
            a_ref[b, r0:, r0:r0 + _TI] = keep.astype(f32)

    ones_col = jnp.ones((_S, 1), f32)
    acc_last = jnp.zeros((_B, _H), f32)
    for k in range(_NBLK):
        for b in range(_B):
            ab = a_ref[b]
            hb = h_ref[b]
            deg = jnp.dot(ab, ones_col,
                          preferred_element_type=f32)
            agg = jnp.dot(ab, hb,
                          preferred_element_type=f32)
            agg = agg / jnp.maximum(deg, 1.0)
            hn = (jnp.dot(agg, wl_ref[k], preferred_element_type=f32)
                  + bl_ref[k]
                  + jnp.dot(hb, wrr_ref[k], preferred_element_type=f32))
            hn_ref[b] = hn
        flat = hn_ref[:].reshape(_B * _S, _H)
        mu = jnp.mean(flat, axis=0, keepdims=True)
        var = jnp.mean((flat - mu) ** 2, axis=0, keepdims=True)
        nb = ((flat - mu) * jax.lax.rsqrt(var + 1e-5) * bng_ref[k]
              + bnb_ref[k])
        h_ref[:] = nb.reshape(_B, _S, _H)
        acc_last = acc_last + h_ref[:, _S - 1, :] * gw_ref[:, k:k + 1]

    z = jnp.maximum(acc_last + gw_ref[:, _NBLK:_NBLK + 1], 0.0)
    he = jnp.maximum(jnp.dot(z, we_ref[:], preferred_element_type=f32)
                     + be_ref[:], 0.0)
    out_ref[:] = (jnp.dot(he, wo_ref[:], preferred_element_type=f32)
                  + bo_ref[:])


def kernel(data, params):
    p = params
    f32 = jnp.float32
    w11 = p['conv11_w'][:, :, 0].T
    b11 = p['conv11_b'].reshape(1, _H)
    w12 = jnp.transpose(p['conv12_w'], (2, 1, 0))
    b12 = p['conv12_b'].reshape(1, _H)
    w21 = p['conv21_w'][:, :, 0].T
    b21 = p['conv21_b'].reshape(1, _H)
    w22 = jnp.transpose(p['conv22_w'], (2, 1, 0))
    b22 = p['conv22_b'].reshape(1, _H)
    w31 = p['conv31_w'][:, :, 0].T
    b31 = p['conv31_b'].reshape(1, _H)
    wf = jnp.stack([p['fc_final_w'][:, :_H].T,
                    p['fc_final_w'][:, _H:2 * _H].T,
                    p['fc_final_w'][:, 2 * _H:].T])
    bf = p['fc_final_b'].reshape(1, _H)
    ws = p['fc1_w'][:, :_H].T
    wr = p['fc1_w'][:, _H:].T
    b1 = p['fc1_b'].reshape(1, _H)
    v = p['fc2_w'][0] - p['fc2_w'][1]
    c = p['fc2_b'][0] - p['fc2_b'][1]
    eye = jnp.eye(_TI, dtype=f32)
    vblk = (eye[:, None, :] * v[None, :, None]).reshape(_TI * _H, _TI)
    g = jax.random.gumbel(jax.random.key(42), (_B, _S * _S, 2), f32)
    gd = (g[..., 0] - g[..., 1]).reshape(_B, _S, _S) + c
    tri = (jnp.arange(_S)[:, None] < jnp.arange(_S)[None, :])
    gd = jnp.swapaxes(jnp.where(tri[None], gd, -jnp.inf), 1, 2)
    wl = jnp.stack([p['sage%d_wl' % k].T for k in range(_NBLK)])
    bl = jnp.stack([p['sage%d_bl' % k].reshape(1, _H) for k in range(_NBLK)])
    wrr = jnp.stack([p['sage%d_wr' % k].T for k in range(_NBLK)])
    bng = jnp.stack([p['bn%d_g' % k].reshape(1, _H) for k in range(_NBLK)])
    bnb = jnp.stack([p['bn%d_b' % k].reshape(1, _H) for k in range(_NBLK)])
    gw = jnp.concatenate([p['gnnw_w'][0], p['gnnw_b']]).reshape(1, 4)
    we = p['fc_extra_w'].T
    be = p['fc_extra_b'].reshape(1, _HE)
    wo = p['out_w'].T
    bo = p['out_b'].reshape(1, _OUT)

    return pl.pallas_call(
        _fused_kernel,
        out_shape=jax.ShapeDtypeStruct((_B, _OUT), f32),
        scratch_shapes=[
            pltpu.VMEM((_B, _S + 2 * _PAD, _H), f32),
            pltpu.VMEM((_B, _S, _H), f32),
            pltpu.VMEM((_B, _S // _TI, _TI * _H), f32),
            pltpu.VMEM((_B, _S, _S), f32),
            pltpu.VMEM((_B, _S, _H), f32),
            pltpu.VMEM((_B, _S, _H), f32),
        ],
    )(data, gd, w11, b11, w12, b12, w21, b21, w22, b22, w31, b31,
      wf, bf, ws, wr, b1, vblk, wl, bl, wrr, bng, bnb, gw, we, be, wo, bo)


# speedup vs baseline: 1.3318x; 1.3318x over previous
import jax
import jax.numpy as jnp
from jax.experimental import pallas as pl
from jax.experimental.pallas import tpu as pltpu

_B, _S, _F, _H, _OUT = 4, 256, 32, 128, 1
_NBLK = 3
_HE = _H // 2
_TI = 8
_PAD = 10
_HI = jax.lax.Precision.HIGHEST


def _fused_kernel(data_ref, gd_ref, w11_ref, b11_ref, w12_ref, b12_ref,
                  w21_ref, b21_ref, w22_ref, b22_ref, w31_ref, b31_ref,
                  wf_ref, bf_ref, ws_ref, wr_ref, b1_ref, vblk_ref,
                  wl_ref, bl_ref, wrr_ref, bng_ref, bnb_ref,
                  gw_ref, we_ref, be_ref, wo_ref, bo_ref,
                  out_ref,
                  pad_ref, ps_ref, pr_ref, a_ref, h_ref, hn_ref):
    f32 = jnp.float32
    dd = data_ref[:].reshape(_B * _S, _F)

    t1 = jnp.dot(dd, w11_ref[:], preferred_element_type=f32, precision=_HI) + b11_ref[:]
    t2 = jnp.dot(dd, w21_ref[:], preferred_element_type=f32, precision=_HI) + b21_ref[:]
    t3 = jnp.dot(dd, w31_ref[:], preferred_element_type=f32, precision=_HI) + b31_ref[:]

    pad_ref[:] = jnp.zeros((_B, _S + 2 * _PAD, _H), f32)
    pad_ref[:, _PAD:_PAD + _S, :] = t1.reshape(_B, _S, _H)
    acc = None
    for k, off in enumerate((-3, 0, 3)):
        sh = pad_ref[:, _PAD + off:_PAD + off + _S, :].reshape(_B * _S, _H)
        p = jnp.dot(sh, w12_ref[k], preferred_element_type=f32, precision=_HI)
        acc = p if acc is None else acc + p
    x1 = acc + b12_ref[:]

    pad_ref[:, _PAD:_PAD + _S, :] = t2.reshape(_B, _S, _H)
    acc = None
    for k, off in enumerate((-10, -5, 0, 5, 10)):
        sh = pad_ref[:, _PAD + off:_PAD + off + _S, :].reshape(_B * _S, _H)
        p = jnp.dot(sh, w22_ref[k], preferred_element_type=f32, precision=_HI)
        acc = p if acc is None else acc + p
    x2 = acc + b22_ref[:]

    x = (jnp.dot(x1, wf_ref[0], preferred_element_type=f32, precision=_HI)
         + jnp.dot(x2, wf_ref[1], preferred_element_type=f32, precision=_HI)
         + jnp.dot(t3, wf_ref[2], preferred_element_type=f32, precision=_HI)
         + bf_ref[:])
    x = jnp.maximum(x, 0.0)

    ps_ref[:] = (jnp.dot(x, ws_ref[:], preferred_element_type=f32, precision=_HI)
                 + b1_ref[:]).reshape(_B, _S, _H)
    pr_ref[:] = jnp.dot(x, wr_ref[:],
                        preferred_element_type=f32,
                        precision=_HI).reshape(_B, _S // _TI,
                                                            _TI * _H)
    h_ref[:] = x.reshape(_B, _S, _H)

    for b in range(_B):
        psb = ps_ref[b]
        tps = jnp.concatenate([psb] * _TI, axis=1)

        def body(it2, _, b=b, tps=tps):
            for u in range(2):
                it = it2 * 2 + u
                prrow = pr_ref[b, pl.ds(it, 1), :]
                ec = jnp.maximum(tps + prrow, 0.0)
                sct = jnp.dot(ec, vblk_ref[:],
                              preferred_element_type=f32)
                scr = sct.T
                keep = scr + gd_ref[b, pl.ds(it * _TI, _TI), :] >= 0.0
                a_ref[b, pl.ds(it * _TI, _TI), :] = keep.astype(f32)
            return 0

        jax.lax.fori_loop(0, _S // (2 * _TI), body, 0)

    ones_col = jnp.ones((_S, 1), f32)
    acc_last = jnp.zeros((_B, _H), f32)
    for k in range(_NBLK):
        for b in range(_B):
            ab = a_ref[b]
            hb = h_ref[b]
            deg = jax.lax.dot_general(ab, ones_col, (((0,), (0,)), ((), ())),
                                      preferred_element_type=f32)
            agg = jax.lax.dot_general(ab, hb, (((0,), (0,)), ((), ())),
                                      preferred_element_type=f32)
            agg = agg / jnp.maximum(deg, 1.0)
            hn = (jnp.dot(agg, wl_ref[k], preferred_element_type=f32)
                  + bl_ref[k]
                  + jnp.dot(hb, wrr_ref[k], preferred_element_type=f32))
            hn_ref[b] = hn
        flat = hn_ref[:].reshape(_B * _S, _H)
        mu = jnp.mean(flat, axis=0, keepdims=True)
        var = jnp.mean((flat - mu) ** 2, axis=0, keepdims=True)
        nb = ((flat - mu) * jax.lax.rsqrt(var + 1e-5) * bng_ref[k]
              + bnb_ref[k])
        h_ref[:] = nb.reshape(_B, _S, _H)
        acc_last = acc_last + h_ref[:, _S - 1, :] * gw_ref[:, k:k + 1]

    z = jnp.maximum(acc_last + gw_ref[:, _NBLK:_NBLK + 1], 0.0)
    he = jnp.maximum(jnp.dot(z, we_ref[:], preferred_element_type=f32)
                     + be_ref[:], 0.0)
    out_ref[:] = (jnp.dot(he, wo_ref[:], preferred_element_type=f32)
                  + bo_ref[:])


def kernel(data, params):
    p = params
    f32 = jnp.float32
    w11 = p['conv11_w'][:, :, 0].T
    b11 = p['conv11_b'].reshape(1, _H)
    w12 = jnp.transpose(p['conv12_w'], (2, 1, 0))
    b12 = p['conv12_b'].reshape(1, _H)
    w21 = p['conv21_w'][:, :, 0].T
    b21 = p['conv21_b'].reshape(1, _H)
    w22 = jnp.transpose(p['conv22_w'], (2, 1, 0))
    b22 = p['conv22_b'].reshape(1, _H)
    w31 = p['conv31_w'][:, :, 0].T
    b31 = p['conv31_b'].reshape(1, _H)
    wf = jnp.stack([p['fc_final_w'][:, :_H].T,
                    p['fc_final_w'][:, _H:2 * _H].T,
                    p['fc_final_w'][:, 2 * _H:].T])
    bf = p['fc_final_b'].reshape(1, _H)
    ws = p['fc1_w'][:, :_H].T
    wr = p['fc1_w'][:, _H:].T
    b1 = p['fc1_b'].reshape(1, _H)
    v = p['fc2_w'][0] - p['fc2_w'][1]
    c = p['fc2_b'][0] - p['fc2_b'][1]
    eye = jnp.eye(_TI, dtype=f32)
    vblk = (eye[:, None, :] * v[None, :, None]).reshape(_TI * _H, _TI)
    g = jax.random.gumbel(jax.random.key(42), (_B, _S * _S, 2), f32)
    gd = (g[..., 0] - g[..., 1]).reshape(_B, _S, _S) + c
    tri = (jnp.arange(_S)[:, None] < jnp.arange(_S)[None, :])
    gd = jnp.where(tri[None], gd, -jnp.inf)
    wl = jnp.stack([p['sage%d_wl' % k].T for k in range(_NBLK)])
    bl = jnp.stack([p['sage%d_bl' % k].reshape(1, _H) for k in range(_NBLK)])
    wrr = jnp.stack([p['sage%d_wr' % k].T for k in range(_NBLK)])
    bng = jnp.stack([p['bn%d_g' % k].reshape(1, _H) for k in range(_NBLK)])
    bnb = jnp.stack([p['bn%d_b' % k].reshape(1, _H) for k in range(_NBLK)])
    gw = jnp.concatenate([p['gnnw_w'][0], p['gnnw_b']]).reshape(1, 4)
    we = p['fc_extra_w'].T
    be = p['fc_extra_b'].reshape(1, _HE)
    wo = p['out_w'].T
    bo = p['out_b'].reshape(1, _OUT)

    return pl.pallas_call(
        _fused_kernel,
        out_shape=jax.ShapeDtypeStruct((_B, _OUT), f32),
        scratch_shapes=[
            pltpu.VMEM((_B, _S + 2 * _PAD, _H), f32),
            pltpu.VMEM((_B, _S, _H), f32),
            pltpu.VMEM((_B, _S // _TI, _TI * _H), f32),
            pltpu.VMEM((_B, _S, _S), f32),
            pltpu.VMEM((_B, _S, _H), f32),
            pltpu.VMEM((_B, _S, _H), f32),
        ],
    )(data, gd, w11, b11, w12, b12, w21, b21, w22, b22, w31, b31,
      wf, bf, ws, wr, b1, vblk, wl, bl, wrr, bng, bnb, gw, we, be, wo, bo)


# speedup vs baseline: 1.3936x; 1.0464x over previous
import jax
import jax.numpy as jnp
from jax.experimental import pallas as pl
from jax.experimental.pallas import tpu as pltpu

_B, _S, _F, _H, _OUT = 4, 256, 32, 128, 1
_NBLK = 3
_HE = _H // 2
_TI = 8
_PAD = 10
_HI = jax.lax.Precision.HIGHEST


def _fused_kernel(data_ref, gd_ref, w11_ref, b11_ref, w12_ref, b12_ref,
                  w21_ref, b21_ref, w22_ref, b22_ref, w31_ref, b31_ref,
                  wf_ref, bf_ref, ws_ref, wr_ref, b1_ref, vblk_ref,
                  wl_ref, bl_ref, wrr_ref, bng_ref, bnb_ref,
                  gw_ref, we_ref, be_ref, wo_ref, bo_ref,
                  out_ref,
                  pad_ref, ps_ref, pr_ref, a_ref, h_ref, hn_ref):
    f32 = jnp.float32
    dd = data_ref[:].reshape(_B * _S, _F)

    t1 = jnp.dot(dd, w11_ref[:], preferred_element_type=f32, precision=_HI) + b11_ref[:]
    t2 = jnp.dot(dd, w21_ref[:], preferred_element_type=f32, precision=_HI) + b21_ref[:]
    t3 = jnp.dot(dd, w31_ref[:], preferred_element_type=f32, precision=_HI) + b31_ref[:]

    pad_ref[:] = jnp.zeros((_B, _S + 2 * _PAD, _H), f32)
    pad_ref[:, _PAD:_PAD + _S, :] = t1.reshape(_B, _S, _H)
    acc = None
    for k, off in enumerate((-3, 0, 3)):
        sh = pad_ref[:, _PAD + off:_PAD + off + _S, :].reshape(_B * _S, _H)
        p = jnp.dot(sh, w12_ref[k], preferred_element_type=f32, precision=_HI)
        acc = p if acc is None else acc + p
    x1 = acc + b12_ref[:]

    pad_ref[:, _PAD:_PAD + _S, :] = t2.reshape(_B, _S, _H)
    acc = None
    for k, off in enumerate((-10, -5, 0, 5, 10)):
        sh = pad_ref[:, _PAD + off:_PAD + off + _S, :].reshape(_B * _S, _H)
        p = jnp.dot(sh, w22_ref[k], preferred_element_type=f32, precision=_HI)
        acc = p if acc is None else acc + p
    x2 = acc + b22_ref[:]

    x = (jnp.dot(x1, wf_ref[0], preferred_element_type=f32, precision=_HI)
         + jnp.dot(x2, wf_ref[1], preferred_element_type=f32, precision=_HI)
         + jnp.dot(t3, wf_ref[2], preferred_element_type=f32, precision=_HI)
         + bf_ref[:])
    x = jnp.maximum(x, 0.0)

    ps_ref[:] = (jnp.dot(x, ws_ref[:], preferred_element_type=f32, precision=_HI)
                 + b1_ref[:]).reshape(_B, _S, _H).astype(jnp.bfloat16)
    pr_ref[:] = jnp.dot(x, wr_ref[:],
                        preferred_element_type=f32,
                        precision=_HI).reshape(_B, _S // _TI, _TI * _H)
    h_ref[:] = x.reshape(_B, _S, _H)

    for b in range(_B):
        psb = ps_ref[b]
        tps = jnp.concatenate([psb] * _TI, axis=1)

        def body(it2, _, b=b, tps=tps):
            for u in range(2):
                it = it2 * 2 + u
                prrow = pr_ref[b, pl.ds(it, 1),
                               :].astype(jnp.bfloat16)
                ec = jnp.maximum(tps + prrow,
                                 jnp.bfloat16(0))
                sct = jnp.dot(ec, vblk_ref[:],
                              preferred_element_type=f32)
                scr = sct.T
                keep = scr + gd_ref[b, pl.ds(it * _TI, _TI), :] >= 0.0
                a_ref[b, pl.ds(it * _TI, _TI), :] = keep.astype(f32)
            return 0

        jax.lax.fori_loop(0, _S // (2 * _TI), body, 0)

    ones_col = jnp.ones((_S, 1), f32)
    acc_last = jnp.zeros((_B, _H), f32)
    for k in range(_NBLK):
        for b in range(_B):
            ab = a_ref[b]
            hb = h_ref[b]
            deg = jax.lax.dot_general(ab, ones_col, (((0,), (0,)), ((), ())),
                                      preferred_element_type=f32)
            agg = jax.lax.dot_general(ab, hb, (((0,), (0,)), ((), ())),
                                      preferred_element_type=f32)
            agg = agg / jnp.maximum(deg, 1.0)
            hn = (jnp.dot(agg, wl_ref[k], preferred_element_type=f32)
                  + bl_ref[k]
                  + jnp.dot(hb, wrr_ref[k], preferred_element_type=f32))
            hn_ref[b] = hn
        flat = hn_ref[:].reshape(_B * _S, _H)
        mu = jnp.mean(flat, axis=0, keepdims=True)
        var = jnp.mean((flat - mu) ** 2, axis=0, keepdims=True)
        nb = ((flat - mu) * jax.lax.rsqrt(var + 1e-5) * bng_ref[k]
              + bnb_ref[k])
        h_ref[:] = nb.reshape(_B, _S, _H)
        acc_last = acc_last + h_ref[:, _S - 1, :] * gw_ref[:, k:k + 1]

    z = jnp.maximum(acc_last + gw_ref[:, _NBLK:_NBLK + 1], 0.0)
    he = jnp.maximum(jnp.dot(z, we_ref[:], preferred_element_type=f32)
                     + be_ref[:], 0.0)
    out_ref[:] = (jnp.dot(he, wo_ref[:], preferred_element_type=f32)
                  + bo_ref[:])


def kernel(data, params):
    p = params
    f32 = jnp.float32
    w11 = p['conv11_w'][:, :, 0].T
    b11 = p['conv11_b'].reshape(1, _H)
    w12 = jnp.transpose(p['conv12_w'], (2, 1, 0))
    b12 = p['conv12_b'].reshape(1, _H)
    w21 = p['conv21_w'][:, :, 0].T
    b21 = p['conv21_b'].reshape(1, _H)
    w22 = jnp.transpose(p['conv22_w'], (2, 1, 0))
    b22 = p['conv22_b'].reshape(1, _H)
    w31 = p['conv31_w'][:, :, 0].T
    b31 = p['conv31_b'].reshape(1, _H)
    wf = jnp.stack([p['fc_final_w'][:, :_H].T,
                    p['fc_final_w'][:, _H:2 * _H].T,
                    p['fc_final_w'][:, 2 * _H:].T])
    bf = p['fc_final_b'].reshape(1, _H)
    ws = p['fc1_w'][:, :_H].T
    wr = p['fc1_w'][:, _H:].T
    b1 = p['fc1_b'].reshape(1, _H)
    v = p['fc2_w'][0] - p['fc2_w'][1]
    c = p['fc2_b'][0] - p['fc2_b'][1]
    eye = jnp.eye(_TI, dtype=f32)
    vblk = (eye[:, None, :] * v[None, :, None]).reshape(_TI * _H,
                                                        _TI).astype(jnp.bfloat16)
    g = jax.random.gumbel(jax.random.key(42), (_B, _S * _S, 2), f32)
    gd = (g[..., 0] - g[..., 1]).reshape(_B, _S, _S) + c
    tri = (jnp.arange(_S)[:, None] < jnp.arange(_S)[None, :])
    gd = jnp.where(tri[None], gd, -jnp.inf)
    wl = jnp.stack([p['sage%d_wl' % k].T for k in range(_NBLK)])
    bl = jnp.stack([p['sage%d_bl' % k].reshape(1, _H) for k in range(_NBLK)])
    wrr = jnp.stack([p['sage%d_wr' % k].T for k in range(_NBLK)])
    bng = jnp.stack([p['bn%d_g' % k].reshape(1, _H) for k in range(_NBLK)])
    bnb = jnp.stack([p['bn%d_b' % k].reshape(1, _H) for k in range(_NBLK)])
    gw = jnp.concatenate([p['gnnw_w'][0], p['gnnw_b']]).reshape(1, 4)
    we = p['fc_extra_w'].T
    be = p['fc_extra_b'].reshape(1, _HE)
    wo = p['out_w'].T
    bo = p['out_b'].reshape(1, _OUT)

    return pl.pallas_call(
        _fused_kernel,
        out_shape=jax.ShapeDtypeStruct((_B, _OUT), f32),
        scratch_shapes=[
            pltpu.VMEM((_B, _S + 2 * _PAD, _H), f32),
            pltpu.VMEM((_B, _S, _H), jnp.bfloat16),
            pltpu.VMEM((_B, _S // _TI, _TI * _H), f32),
            pltpu.VMEM((_B, _S, _S), f32),
            pltpu.VMEM((_B, _S, _H), f32),
            pltpu.VMEM((_B, _S, _H), f32),
        ],
    )(data, gd, w11, b11, w12, b12, w21, b21, w22, b22, w31, b31,
      wf, bf, ws, wr, b1, vblk, wl, bl, wrr, bng, bnb, gw, we, be, wo, bo)


# speedup vs baseline: 1.4837x; 1.0647x over previous
import jax
import jax.numpy as jnp
from jax.experimental import pallas as pl
from jax.experimental.pallas import tpu as pltpu

_B, _S, _F, _H, _OUT = 4, 256, 32, 128, 1
_NBLK = 3
_HE = _H // 2
_TI = 8
_PAD = 10
_HI = jax.lax.Precision.HIGHEST


def _fused_kernel(data_ref, gd_ref, w11_ref, b11_ref, w12_ref, b12_ref,
                  w21_ref, b21_ref, w22_ref, b22_ref, w31_ref, b31_ref,
                  wf_ref, bf_ref, ws_ref, wr_ref, b1_ref, vblk_ref,
                  wl_ref, bl_ref, wrr_ref, bng_ref, bnb_ref,
                  gw_ref, we_ref, be_ref, wo_ref, bo_ref,
                  out_ref,
                  pad_ref, ps_ref, pr_ref, a_ref, h_ref, hn_ref):
    f32 = jnp.float32
    dd = data_ref[:].reshape(_B * _S, _F)

    t1 = jnp.dot(dd, w11_ref[:], preferred_element_type=f32, precision=_HI) + b11_ref[:]
    t2 = jnp.dot(dd, w21_ref[:], preferred_element_type=f32, precision=_HI) + b21_ref[:]
    t3 = jnp.dot(dd, w31_ref[:], preferred_element_type=f32, precision=_HI) + b31_ref[:]

    pad_ref[:] = jnp.zeros((_B, _S + 2 * _PAD, _H), f32)
    pad_ref[:, _PAD:_PAD + _S, :] = t1.reshape(_B, _S, _H)
    acc = None
    for k, off in enumerate((-3, 0, 3)):
        sh = pad_ref[:, _PAD + off:_PAD + off + _S, :].reshape(_B * _S, _H)
        p = jnp.dot(sh, w12_ref[k], preferred_element_type=f32, precision=_HI)
        acc = p if acc is None else acc + p
    x1 = acc + b12_ref[:]

    pad_ref[:, _PAD:_PAD + _S, :] = t2.reshape(_B, _S, _H)
    acc = None
    for k, off in enumerate((-10, -5, 0, 5, 10)):
        sh = pad_ref[:, _PAD + off:_PAD + off + _S, :].reshape(_B * _S, _H)
        p = jnp.dot(sh, w22_ref[k], preferred_element_type=f32, precision=_HI)
        acc = p if acc is None else acc + p
    x2 = acc + b22_ref[:]

    x = (jnp.dot(x1, wf_ref[0], preferred_element_type=f32, precision=_HI)
         + jnp.dot(x2, wf_ref[1], preferred_element_type=f32, precision=_HI)
         + jnp.dot(t3, wf_ref[2], preferred_element_type=f32, precision=_HI)
         + bf_ref[:])
    x = jnp.maximum(x, 0.0)

    ps_ref[:] = (jnp.dot(x, ws_ref[:], preferred_element_type=f32, precision=_HI)
                 + b1_ref[:]).reshape(_B, _S, _H).astype(jnp.bfloat16)
    pr_ref[:] = jnp.dot(x, wr_ref[:],
                        preferred_element_type=f32,
                        precision=_HI).reshape(_B, _S // _TI, _TI * _H)
    h_ref[:] = x.reshape(_B, _S, _H)

    half = _S // 2
    for b in range(_B):
        psb = ps_ref[b]
        tps = jnp.concatenate([psb] * _TI, axis=1)
        tpsu = tps[half:]

        def body(it2, _, b=b, tps=tps):
            for u in range(2):
                it = it2 * 2 + u
                prrow = pr_ref[b, pl.ds(it, 1),
                               :].astype(jnp.bfloat16)
                ec = jnp.maximum(tps + prrow,
                                 jnp.bfloat16(0))
                sct = jnp.dot(ec, vblk_ref[:],
                              preferred_element_type=f32)
                scr = sct.T
                keep = scr + gd_ref[b, pl.ds(it * _TI, _TI), :] >= 0.0
                a_ref[b, pl.ds(it * _TI, _TI), :] = keep.astype(f32)
            return 0

        jax.lax.fori_loop(0, half // (2 * _TI), body, 0)

        def bodyu(it2, _, b=b, tpsu=tpsu):
            for u in range(2):
                it = half // _TI + it2 * 2 + u
                prrow = pr_ref[b, pl.ds(it, 1),
                               :].astype(jnp.bfloat16)
                ec = jnp.maximum(tpsu + prrow,
                                 jnp.bfloat16(0))
                sct = jnp.dot(ec, vblk_ref[:],
                              preferred_element_type=f32)
                scr = sct.T
                keep = scr + gd_ref[b, pl.ds(it * _TI, _TI), half:] >= 0.0
                a_ref[b, pl.ds(it * _TI, _TI), half:] = keep.astype(f32)
            return 0

        a_ref[b, half:, :half] = jnp.zeros((half, half), f32)
        jax.lax.fori_loop(0, half // (2 * _TI), bodyu, 0)

    ones_col = jnp.ones((_S, 1), f32)
    acc_last = jnp.zeros((_B, _H), f32)
    for k in range(_NBLK):
        for b in range(_B):
            ab = a_ref[b]
            hb = h_ref[b]
            deg = jax.lax.dot_general(ab, ones_col, (((0,), (0,)), ((), ())),
                                      preferred_element_type=f32)
            agg = jax.lax.dot_general(ab, hb, (((0,), (0,)), ((), ())),
                                      preferred_element_type=f32)
            agg = agg / jnp.maximum(deg, 1.0)
            hn = (jnp.dot(agg, wl_ref[k], preferred_element_type=f32)
                  + bl_ref[k]
                  + jnp.dot(hb, wrr_ref[k], preferred_element_type=f32))
            hn_ref[b] = hn
        flat = hn_ref[:].reshape(_B * _S, _H)
        mu = jnp.mean(flat, axis=0, keepdims=True)
        var = jnp.mean((flat - mu) ** 2, axis=0, keepdims=True)
        nb = ((flat - mu) * jax.lax.rsqrt(var + 1e-5) * bng_ref[k]
              + bnb_ref[k])
        h_ref[:] = nb.reshape(_B, _S, _H)
        acc_last = acc_last + h_ref[:, _S - 1, :] * gw_ref[:, k:k + 1]

    z = jnp.maximum(acc_last + gw_ref[:, _NBLK:_NBLK + 1], 0.0)
    he = jnp.maximum(jnp.dot(z, we_ref[:], preferred_element_type=f32)
                     + be_ref[:], 0.0)
    out_ref[:] = (jnp.dot(he, wo_ref[:], preferred_element_type=f32)
                  + bo_ref[:])


def kernel(data, params):
    p = params
    f32 = jnp.float32
    w11 = p['conv11_w'][:, :, 0].T
    b11 = p['conv11_b'].reshape(1, _H)
    w12 = jnp.transpose(p['conv12_w'], (2, 1, 0))
    b12 = p['conv12_b'].reshape(1, _H)
    w21 = p['conv21_w'][:, :, 0].T
    b21 = p['conv21_b'].reshape(1, _H)
    w22 = jnp.transpose(p['conv22_w'], (2, 1, 0))
    b22 = p['conv22_b'].reshape(1, _H)
    w31 = p['conv31_w'][:, :, 0].T
    b31 = p['conv31_b'].reshape(1, _H)
    wf = jnp.stack([p['fc_final_w'][:, :_H].T,
                    p['fc_final_w'][:, _H:2 * _H].T,
                    p['fc_final_w'][:, 2 * _H:].T])
    bf = p['fc_final_b'].reshape(1, _H)
    ws = p['fc1_w'][:, :_H].T
    wr = p['fc1_w'][:, _H:].T
    b1 = p['fc1_b'].reshape(1, _H)
    v = p['fc2_w'][0] - p['fc2_w'][1]
    c = p['fc2_b'][0] - p['fc2_b'][1]
    eye = jnp.eye(_TI, dtype=f32)
    vblk = (eye[:, None, :] * v[None, :, None]).reshape(_TI * _H,
                                                        _TI).astype(jnp.bfloat16)
    g = jax.random.gumbel(jax.random.key(42), (_B, _S * _S, 2), f32)
    gd = (g[..., 0] - g[..., 1]).reshape(_B, _S, _S) + c
    tri = (jnp.arange(_S)[:, None] < jnp.arange(_S)[None, :])
    gd = jnp.where(tri[None], gd, -jnp.inf)
    wl = jnp.stack([p['sage%d_wl' % k].T for k in range(_NBLK)])
    bl = jnp.stack([p['sage%d_bl' % k].reshape(1, _H) for k in range(_NBLK)])
    wrr = jnp.stack([p['sage%d_wr' % k].T for k in range(_NBLK)])
    bng = jnp.stack([p['bn%d_g' % k].reshape(1, _H) for k in range(_NBLK)])
    bnb = jnp.stack([p['bn%d_b' % k].reshape(1, _H) for k in range(_NBLK)])
    gw = jnp.concatenate([p['gnnw_w'][0], p['gnnw_b']]).reshape(1, 4)
    we = p['fc_extra_w'].T
    be = p['fc_extra_b'].reshape(1, _HE)
    wo = p['out_w'].T
    bo = p['out_b'].reshape(1, _OUT)

    return pl.pallas_call(
        _fused_kernel,
        out_shape=jax.ShapeDtypeStruct((_B, _OUT), f32),
        scratch_shapes=[
            pltpu.VMEM((_B, _S + 2 * _PAD, _H), f32),
            pltpu.VMEM((_B, _S, _H), jnp.bfloat16),
            pltpu.VMEM((_B, _S // _TI, _TI * _H), f32),
            pltpu.VMEM((_B, _S, _S), f32),
            pltpu.VMEM((_B, _S, _H), f32),
            pltpu.VMEM((_B, _S, _H), f32),
        ],
    )(data, gd, w11, b11, w12, b12, w21, b21, w22, b22, w31, b31,
      wf, bf, ws, wr, b1, vblk, wl, bl, wrr, bng, bnb, gw, we, be, wo, bo)
